# baseline (speedup 1.0000x reference)
import jax
import jax.numpy as jnp
from jax import lax
from jax.experimental import pallas as pl
from jax.experimental.pallas import tpu as pltpu

INPUT_DIM = 2048
HIDDEN_DIM = 16384
TOPK = 64
BATCH = 2048

BH = 512
TB = 256
KB = 1024
DB = 1024


def _encode_body(x_ref, w_ref, b_ref, out_ref):
    xm = (x_ref[...] - b_ref[...][None, :]).astype(jnp.bfloat16)
    out_ref[...] = jax.lax.dot_general(
        xm, w_ref[...].astype(jnp.bfloat16),
        dimension_numbers=(((1,), (1,)), ((), ())),
        preferred_element_type=jnp.float32,
    )


def _encode(x, W_enc, bias):
    return pl.pallas_call(
        _encode_body,
        grid=(HIDDEN_DIM // BH,),
        in_specs=[
            pl.BlockSpec((BATCH, INPUT_DIM), lambda h: (0, 0)),
            pl.BlockSpec((BH, INPUT_DIM), lambda h: (h, 0)),
            pl.BlockSpec((INPUT_DIM,), lambda h: (0,)),
        ],
        out_specs=pl.BlockSpec((BATCH, BH), lambda h: (0, h)),
        out_shape=jax.ShapeDtypeStruct((BATCH, HIDDEN_DIM), jnp.float32),
    )(x, W_enc, bias)


def _u32_to_f32(t):
    neg = (t & jnp.uint32(0x80000000)) == 0
    bits = jnp.where(neg, ~t, t & jnp.uint32(0x7FFFFFFF))
    return lax.bitcast_convert_type(bits, jnp.float32)


def _f32_to_key(x):
    k = lax.bitcast_convert_type(x, jnp.int32)
    u = lax.bitcast_convert_type(k, jnp.uint32)
    return jnp.where(k < 0, ~u, u | jnp.uint32(0x80000000))


def _threshold_body(pre_ref, t_ref):
    pre = pre_ref[...]

    cm = pre[:, :128]
    for c in range(1, HIDDEN_DIM // 128):
        cm = jnp.maximum(cm, pre[:, c * 128:(c + 1) * 128])
    himax = _f32_to_key(jnp.max(cm, axis=1, keepdims=True))

    def step_cm(_, carry):
        lo, hi = carry
        mid = lo + ((hi - lo + jnp.uint32(1)) >> jnp.uint32(1))
        t_f = _u32_to_f32(mid)
        cnt = jnp.sum((cm >= t_f).astype(jnp.int32), axis=1, keepdims=True)
        take = cnt >= TOPK
        lo = jnp.where(take, mid, lo)
        hi = jnp.where(take, hi, mid - jnp.uint32(1))
        return lo, hi

    lo_cm, _ = lax.fori_loop(
        0, 32, step_cm, (jnp.zeros((TB, 1), jnp.uint32), himax))

    def cond(carry):
        lo, hi = carry
        return jnp.any(lo < hi)

    def step(carry):
        lo, hi = carry
        mid = lo + ((hi - lo + jnp.uint32(1)) >> jnp.uint32(1))
        t_f = _u32_to_f32(mid)
        cnt = jnp.sum((pre >= t_f).astype(jnp.int32), axis=1, keepdims=True)
        take = cnt >= TOPK
        lo = jnp.where(take, mid, lo)
        hi = jnp.where(take, hi, mid - jnp.uint32(1))
        return lo, hi

    lo, _ = lax.while_loop(cond, step, (lo_cm, himax))
    t_ref[...] = jnp.broadcast_to(_u32_to_f32(lo), (TB, 128))


def _threshold(pre):
    return pl.pallas_call(
        _threshold_body,
        grid=(BATCH // TB,),
        in_specs=[pl.BlockSpec((TB, HIDDEN_DIM), lambda b: (b, 0))],
        out_specs=pl.BlockSpec((TB, 128), lambda b: (b, 0)),
        out_shape=jax.ShapeDtypeStruct((BATCH, 128), jnp.float32),
    )(pre)


def _decode_body(pre_ref, t_ref, w_ref, b_ref, out_ref):
    k = pl.program_id(1)
    t = t_ref[...][:, :1]
    p = pre_ref[...]
    z = jnp.where(p >= t, jnp.maximum(p, 0.0), 0.0).astype(jnp.bfloat16)
    acc = jax.lax.dot_general(
        z, w_ref[...].astype(jnp.bfloat16),
        dimension_numbers=(((1,), (1,)), ((), ())),
        preferred_element_type=jnp.float32,
    )

    @pl.when(k == 0)
    def _():
        out_ref[...] = acc + b_ref[...][None, :]

    @pl.when(k > 0)
    def _():
        out_ref[...] += acc


def _masked_decode(pre, trep, W_dec, bias):
    return pl.pallas_call(
        _decode_body,
        grid=(BATCH // DB, HIDDEN_DIM // KB),
        in_specs=[
            pl.BlockSpec((DB, KB), lambda b, k: (b, k)),
            pl.BlockSpec((DB, 128), lambda b, k: (b, 0)),
            pl.BlockSpec((INPUT_DIM, KB), lambda b, k: (0, k)),
            pl.BlockSpec((INPUT_DIM,), lambda b, k: (0,)),
        ],
        out_specs=pl.BlockSpec((DB, INPUT_DIM), lambda b, k: (b, 0)),
        out_shape=jax.ShapeDtypeStruct((BATCH, INPUT_DIM), jnp.float32),
    )(pre, trep, W_dec, bias)


NC = 2
NS = 16
NW = NC * NS
ROWS_PER_W = BATCH // NW
L = 16
NB = 8
GCHUNK = 16
NCHUNK = TOPK // GCHUNK
VTILE = 32
NTILE = INPUT_DIM // (VTILE * L)


def _take(v, idx):
    return jnp.take_along_axis(v, idx, axis=0, mode="promise_in_bounds")


def _sc_body(pre_hbm, trep_hbm, w_hbm, bias_hbm, out_hbm,
             pre2, t_v, valbuf, idxbuf, valrep, rows2,
             acc_v, bias_v, sem_pre, sem_g, sem_out):
    wid = lax.axis_index("s") * NC + lax.axis_index("c")
    base_row = wid * ROWS_PER_W

    pltpu.sync_copy(bias_hbm, bias_v)
    pltpu.sync_copy(trep_hbm.at[pl.ds(base_row, ROWS_PER_W)], t_v)
    pltpu.async_copy(pre_hbm.at[base_row], pre2.at[0], sem_pre)
    lane = lax.iota(jnp.int32, L)
    ones = jnp.ones((L,), jnp.float32)
    perms = [lax.rem(lane + l, L) for l in range(L)]
    folds = [jnp.bitwise_xor(lane, s) for s in (8, 4, 2, 1)]

    def _fold_max(x):
        for f in folds:
            x = jnp.maximum(x, _take(x, f))
        return x[0]

    def do_row(r, carry):
        row = base_row + r
        slot = lax.rem(r, 2)
        nslot = lax.rem(r + 1, 2)
        pltpu.make_async_copy(pre_hbm.at[row], pre2.at[slot], sem_pre).wait()

        @pl.when(r < ROWS_PER_W - 1)
        def _():
            pltpu.async_copy(pre_hbm.at[row + 1], pre2.at[nslot], sem_pre)

        tv = t_v[r, :]
        ts = tv[0]

        def batch(i, off):
            mx = pre2[slot, pl.ds(i * (NB * L), L)]
            for q in range(1, NB):
                mx = jnp.maximum(mx, pre2[slot, pl.ds(i * (NB * L) + q * L, L)])

            def hit_batch(off):
                for q in range(NB):
                    v = pre2[slot, pl.ds(i * (NB * L) + q * L, L)]
                    mi = jnp.where(v >= tv, jnp.int32(1), jnp.int32(0))

                    def hit_vreg(off, v=v, mi=mi, q=q):
                        iv = lane + (i * (NB * L) + q * L)

                        def put(off, l=0):
                            pass
                        for l in range(L):
                            def put_l(off, l=l, v=v, iv=iv):
                                valbuf[pl.ds(off, L)] = _take(v, perms[l])
                                idxbuf[pl.ds(off, L)] = _take(iv, perms[l])
                                return off + 1
                            off = lax.cond(
                                jnp.logical_and(mi[l] > 0, off < TOPK),
                                put_l, lambda o: o, off)
                        return off

                    off = lax.cond(_fold_max(mi) > 0, hit_vreg,
                                   lambda o: o, off)
                return off

            return lax.cond(_fold_max(mx) >= ts, hit_batch,
                            lambda o: o, off)

        lax.fori_loop(0, HIDDEN_DIM // (NB * L), batch, jnp.int32(0))

        for c in range(TOPK // L):
            vb = jnp.maximum(valbuf[pl.ds(c * L, L)], 0.0)
            for j in range(L):
                valrep[c * L + j, :] = vb[j] * ones

        pltpu.async_copy(w_hbm.at[idxbuf.at[pl.ds(0, GCHUNK)]],
                         rows2.at[0], sem_g)
        for c in range(NCHUNK):
            gslot = c % 2
            pltpu.make_async_copy(w_hbm.at[idxbuf.at[pl.ds(c * GCHUNK, GCHUNK)]],
                                  rows2.at[gslot], sem_g).wait()
            if c < NCHUNK - 1:
                pltpu.async_copy(
                    w_hbm.at[idxbuf.at[pl.ds((c + 1) * GCHUNK, GCHUNK)]],
                    rows2.at[(c + 1) % 2], sem_g)
            for t in range(NTILE):
                tbase = t * VTILE * L
                if c == 0:
                    acc = tuple(bias_v[pl.ds(tbase + i2 * L, L)]
                                for i2 in range(VTILE))
                else:
                    acc = tuple(acc_v[pl.ds(tbase + i2 * L, L)]
                                for i2 in range(VTILE))

                def mac(j, a, c=c, gslot=gslot, tbase=tbase):
                    sv = valrep[c * GCHUNK + j, :]
                    return tuple(
                        a[i2] + sv * rows2[gslot, j, pl.ds(tbase + i2 * L, L)]
                        for i2 in range(VTILE))

                acc = lax.fori_loop(0, GCHUNK, mac, acc)
                for i2 in range(VTILE):
                    acc_v[pl.ds(tbase + i2 * L, L)] = acc[i2]

        pltpu.async_copy(acc_v, out_hbm.at[row], sem_out).wait()
        return carry

    lax.fori_loop(0, ROWS_PER_W, do_row, 0)


def _sc_topk_decode(pre, trep, W_enc, bias):
    from jax.experimental.pallas import tpu_sc as plsc
    mesh = plsc.VectorSubcoreMesh(core_axis_name="c", subcore_axis_name="s")
    k = pl.kernel(
        _sc_body,
        mesh=mesh,
        out_type=jax.ShapeDtypeStruct((BATCH, INPUT_DIM), jnp.float32),
        scratch_types=[
            pltpu.VMEM((2, HIDDEN_DIM), jnp.float32),
            pltpu.VMEM((ROWS_PER_W, L), jnp.float32),
            pltpu.VMEM((TOPK + L,), jnp.float32),
            pltpu.VMEM((TOPK + L,), jnp.int32),
            pltpu.VMEM((TOPK, L), jnp.float32),
            pltpu.VMEM((2, GCHUNK, INPUT_DIM), jnp.float32),
            pltpu.VMEM((INPUT_DIM,), jnp.float32),
            pltpu.VMEM((INPUT_DIM,), jnp.float32),
            pltpu.SemaphoreType.DMA,
            pltpu.SemaphoreType.DMA,
            pltpu.SemaphoreType.DMA,
        ],
    )
    return k(pre, trep, W_enc, bias)


def kernel(x, W_enc, W_dec, bias):
    pre = _encode(x, W_enc, bias)
    trep = _threshold(pre)
    return _sc_topk_decode(pre, trep[:, :L], W_enc, bias)


# speedup vs baseline: 2.6112x; 2.6112x over previous
import jax
import jax.numpy as jnp
from jax import lax
from jax.experimental import pallas as pl
from jax.experimental.pallas import tpu as pltpu

INPUT_DIM = 2048
HIDDEN_DIM = 16384
TOPK = 64
BATCH = 2048

BH = 512
TB = 256
KB = 1024


def _encode_body(x_ref, w_ref, b_ref, out_ref):
    xm = (x_ref[...] - b_ref[...][None, :]).astype(jnp.bfloat16)
    out_ref[...] = jax.lax.dot_general(
        xm, w_ref[...].astype(jnp.bfloat16),
        dimension_numbers=(((1,), (1,)), ((), ())),
        preferred_element_type=jnp.float32,
    )


def _encode(x, W_enc, bias):
    return pl.pallas_call(
        _encode_body,
        grid=(HIDDEN_DIM // BH,),
        in_specs=[
            pl.BlockSpec((BATCH, INPUT_DIM), lambda h: (0, 0)),
            pl.BlockSpec((BH, INPUT_DIM), lambda h: (h, 0)),
            pl.BlockSpec((INPUT_DIM,), lambda h: (0,)),
        ],
        out_specs=pl.BlockSpec((BATCH, BH), lambda h: (0, h)),
        out_shape=jax.ShapeDtypeStruct((BATCH, HIDDEN_DIM), jnp.float32),
    )(x, W_enc, bias)


def _u32_to_f32(t):
    neg = (t & jnp.uint32(0x80000000)) == 0
    bits = jnp.where(neg, ~t, t & jnp.uint32(0x7FFFFFFF))
    return lax.bitcast_convert_type(bits, jnp.float32)


def _f32_to_key(x):
    k = lax.bitcast_convert_type(x, jnp.int32)
    u = lax.bitcast_convert_type(k, jnp.uint32)
    return jnp.where(k < 0, ~u, u | jnp.uint32(0x80000000))


def _threshold_body(pre_ref, t_ref):
    pre = pre_ref[...]

    cm = pre[:, :128]
    for c in range(1, HIDDEN_DIM // 128):
        cm = jnp.maximum(cm, pre[:, c * 128:(c + 1) * 128])
    himax = _f32_to_key(jnp.max(cm, axis=1, keepdims=True))

    def step_cm(_, carry):
        lo, hi = carry
        mid = lo + ((hi - lo + jnp.uint32(1)) >> jnp.uint32(1))
        t_f = _u32_to_f32(mid)
        cnt = jnp.sum((cm >= t_f).astype(jnp.int32), axis=1, keepdims=True)
        take = cnt >= TOPK
        lo = jnp.where(take, mid, lo)
        hi = jnp.where(take, hi, mid - jnp.uint32(1))
        return lo, hi

    lo_cm, _ = lax.fori_loop(
        0, 32, step_cm, (jnp.zeros((TB, 1), jnp.uint32), himax))

    def cond(carry):
        lo, hi = carry
        return jnp.any(lo < hi)

    def step(carry):
        lo, hi = carry
        mid = lo + ((hi - lo + jnp.uint32(1)) >> jnp.uint32(1))
        t_f = _u32_to_f32(mid)
        cnt = jnp.sum((pre >= t_f).astype(jnp.int32), axis=1, keepdims=True)
        take = cnt >= TOPK
        lo = jnp.where(take, mid, lo)
        hi = jnp.where(take, hi, mid - jnp.uint32(1))
        return lo, hi

    lo, _ = lax.while_loop(cond, step, (lo_cm, himax))
    t_ref[...] = jnp.broadcast_to(_u32_to_f32(lo), (TB, 128))


def _threshold(pre):
    return pl.pallas_call(
        _threshold_body,
        grid=(BATCH // TB,),
        in_specs=[pl.BlockSpec((TB, HIDDEN_DIM), lambda b: (b, 0))],
        out_specs=pl.BlockSpec((TB, 128), lambda b: (b, 0)),
        out_shape=jax.ShapeDtypeStruct((BATCH, 128), jnp.float32),
    )(pre)


def _decode_body(pre_ref, t_ref, w_ref, b_ref, out_ref):
    k = pl.program_id(1)
    t = t_ref[...][:, :1]
    p = pre_ref[...]
    z = jnp.where(p >= t, jnp.maximum(p, 0.0), 0.0).astype(jnp.bfloat16)
    acc = jax.lax.dot_general(
        z, w_ref[...].astype(jnp.bfloat16),
        dimension_numbers=(((1,), (1,)), ((), ())),
        preferred_element_type=jnp.float32,
    )

    @pl.when(k == 0)
    def _():
        out_ref[...] = acc + b_ref[...][None, :]

    @pl.when(k > 0)
    def _():
        out_ref[...] += acc


def _masked_decode(pre, trep, W_dec, bias):
    nb = pre.shape[0]
    return pl.pallas_call(
        _decode_body,
        grid=(nb // DB, HIDDEN_DIM // KB),
        in_specs=[
            pl.BlockSpec((DB, KB), lambda b, k: (b, k)),
            pl.BlockSpec((DB, 128), lambda b, k: (b, 0)),
            pl.BlockSpec((INPUT_DIM, KB), lambda b, k: (0, k)),
            pl.BlockSpec((INPUT_DIM,), lambda b, k: (0,)),
        ],
        out_specs=pl.BlockSpec((DB, INPUT_DIM), lambda b, k: (b, 0)),
        out_shape=jax.ShapeDtypeStruct((nb, INPUT_DIM), jnp.float32),
    )(pre, trep, W_dec, bias)


NC = 2
NS = 16
NW = NC * NS
SC_BATCH = 256
ROWS_PER_W = SC_BATCH // NW
L = 16
NB = 8
GCHUNK = 16
NCHUNK = TOPK // GCHUNK
VTILE = 32
NTILE = INPUT_DIM // (VTILE * L)


def _take(v, idx):
    return jnp.take_along_axis(v, idx, axis=0, mode="promise_in_bounds")


def _sc_body(pre_hbm, trep_hbm, w_hbm, bias_hbm, out_hbm,
             pre2, t_v, valbuf, idxbuf, valrep, rows2,
             acc_v, bias_v, sem_pre, sem_g, sem_out):
    wid = lax.axis_index("s") * NC + lax.axis_index("c")
    base_row = wid * ROWS_PER_W

    pltpu.sync_copy(bias_hbm, bias_v)
    pltpu.sync_copy(trep_hbm.at[pl.ds(base_row, ROWS_PER_W)], t_v)
    pltpu.async_copy(pre_hbm.at[base_row], pre2.at[0], sem_pre)
    lane = lax.iota(jnp.int32, L)
    ones = jnp.ones((L,), jnp.float32)
    perms = [lax.rem(lane + l, L) for l in range(L)]
    folds = [jnp.bitwise_xor(lane, s) for s in (8, 4, 2, 1)]

    def _fold_max(x):
        for f in folds:
            x = jnp.maximum(x, _take(x, f))
        return x[0]

    def do_row(r, carry):
        row = base_row + r
        slot = lax.rem(r, 2)
        nslot = lax.rem(r + 1, 2)
        pltpu.make_async_copy(pre_hbm.at[row], pre2.at[slot], sem_pre).wait()

        @pl.when(r < ROWS_PER_W - 1)
        def _():
            pltpu.async_copy(pre_hbm.at[row + 1], pre2.at[nslot], sem_pre)

        tv = t_v[r, :]
        ts = tv[0]

        def batch(i, off):
            mx = pre2[slot, pl.ds(i * (NB * L), L)]
            for q in range(1, NB):
                mx = jnp.maximum(mx, pre2[slot, pl.ds(i * (NB * L) + q * L, L)])

            def hit_batch(off):
                for q in range(NB):
                    v = pre2[slot, pl.ds(i * (NB * L) + q * L, L)]
                    mi = jnp.where(v >= tv, jnp.int32(1), jnp.int32(0))

                    def hit_vreg(off, v=v, mi=mi, q=q):
                        iv = lane + (i * (NB * L) + q * L)

                        def put(off, l=0):
                            pass
                        for l in range(L):
                            def put_l(off, l=l, v=v, iv=iv):
                                valbuf[pl.ds(off, L)] = _take(v, perms[l])
                                idxbuf[pl.ds(off, L)] = _take(iv, perms[l])
                                return off + 1
                            off = lax.cond(
                                jnp.logical_and(mi[l] > 0, off < TOPK),
                                put_l, lambda o: o, off)
                        return off

                    off = lax.cond(_fold_max(mi) > 0, hit_vreg,
                                   lambda o: o, off)
                return off

            return lax.cond(_fold_max(mx) >= ts, hit_batch,
                            lambda o: o, off)

        lax.fori_loop(0, HIDDEN_DIM // (NB * L), batch, jnp.int32(0))

        for c in range(TOPK // L):
            vb = jnp.maximum(valbuf[pl.ds(c * L, L)], 0.0)
            for j in range(L):
                valrep[c * L + j, :] = vb[j] * ones

        pltpu.async_copy(w_hbm.at[idxbuf.at[pl.ds(0, GCHUNK)]],
                         rows2.at[0], sem_g)
        for c in range(NCHUNK):
            gslot = c % 2
            pltpu.make_async_copy(w_hbm.at[idxbuf.at[pl.ds(c * GCHUNK, GCHUNK)]],
                                  rows2.at[gslot], sem_g).wait()
            if c < NCHUNK - 1:
                pltpu.async_copy(
                    w_hbm.at[idxbuf.at[pl.ds((c + 1) * GCHUNK, GCHUNK)]],
                    rows2.at[(c + 1) % 2], sem_g)
            for t in range(NTILE):
                tbase = t * VTILE * L
                if c == 0:
                    acc = tuple(bias_v[pl.ds(tbase + i2 * L, L)]
                                for i2 in range(VTILE))
                else:
                    acc = tuple(acc_v[pl.ds(tbase + i2 * L, L)]
                                for i2 in range(VTILE))

                def mac(j, a, c=c, gslot=gslot, tbase=tbase):
                    sv = valrep[c * GCHUNK + j, :]
                    return tuple(
                        a[i2] + sv * rows2[gslot, j, pl.ds(tbase + i2 * L, L)]
                        for i2 in range(VTILE))

                acc = lax.fori_loop(0, GCHUNK, mac, acc, unroll=4)
                for i2 in range(VTILE):
                    acc_v[pl.ds(tbase + i2 * L, L)] = acc[i2]

        pltpu.async_copy(acc_v, out_hbm.at[row], sem_out).wait()
        return carry

    lax.fori_loop(0, ROWS_PER_W, do_row, 0)


def _sc_topk_decode(pre, trep, W_enc, bias):
    from jax.experimental.pallas import tpu_sc as plsc
    mesh = plsc.VectorSubcoreMesh(core_axis_name="c", subcore_axis_name="s")
    k = pl.kernel(
        _sc_body,
        mesh=mesh,
        out_type=jax.ShapeDtypeStruct((SC_BATCH, INPUT_DIM), jnp.float32),
        scratch_types=[
            pltpu.VMEM((2, HIDDEN_DIM), jnp.float32),
            pltpu.VMEM((ROWS_PER_W, L), jnp.float32),
            pltpu.VMEM((TOPK + L,), jnp.float32),
            pltpu.VMEM((TOPK + L,), jnp.int32),
            pltpu.VMEM((TOPK, L), jnp.float32),
            pltpu.VMEM((2, GCHUNK, INPUT_DIM), jnp.float32),
            pltpu.VMEM((INPUT_DIM,), jnp.float32),
            pltpu.VMEM((INPUT_DIM,), jnp.float32),
            pltpu.SemaphoreType.DMA,
            pltpu.SemaphoreType.DMA,
            pltpu.SemaphoreType.DMA,
        ],
    )
    return k(pre, trep, W_enc, bias)


TC_BATCH = BATCH - SC_BATCH
DB = 896


def kernel(x, W_enc, W_dec, bias):
    pre = _encode(x, W_enc, bias)
    trep = _threshold(pre)
    out_tc = _masked_decode(pre[:TC_BATCH], trep[:TC_BATCH], W_dec, bias)
    out_sc = _sc_topk_decode(pre[TC_BATCH:], trep[TC_BATCH:, :L], W_enc, bias)
    return jnp.concatenate([out_tc, out_sc], axis=0)


# speedup vs baseline: 2.6142x; 1.0012x over previous
import jax
import jax.numpy as jnp
from jax import lax
from jax.experimental import pallas as pl
from jax.experimental.pallas import tpu as pltpu

INPUT_DIM = 2048
HIDDEN_DIM = 16384
TOPK = 64
BATCH = 2048

BH = 512
TB = 256
KB = 1024


def _encode_body(x_ref, w_ref, b_ref, out_ref):
    xm = (x_ref[...] - b_ref[...][None, :]).astype(jnp.bfloat16)
    out_ref[...] = jax.lax.dot_general(
        xm, w_ref[...].astype(jnp.bfloat16),
        dimension_numbers=(((1,), (1,)), ((), ())),
        preferred_element_type=jnp.float32,
    )


def _encode(x, W_enc, bias):
    return pl.pallas_call(
        _encode_body,
        grid=(HIDDEN_DIM // BH,),
        in_specs=[
            pl.BlockSpec((BATCH, INPUT_DIM), lambda h: (0, 0)),
            pl.BlockSpec((BH, INPUT_DIM), lambda h: (h, 0)),
            pl.BlockSpec((INPUT_DIM,), lambda h: (0,)),
        ],
        out_specs=pl.BlockSpec((BATCH, BH), lambda h: (0, h)),
        out_shape=jax.ShapeDtypeStruct((BATCH, HIDDEN_DIM), jnp.float32),
    )(x, W_enc, bias)


def _u32_to_f32(t):
    neg = (t & jnp.uint32(0x80000000)) == 0
    bits = jnp.where(neg, ~t, t & jnp.uint32(0x7FFFFFFF))
    return lax.bitcast_convert_type(bits, jnp.float32)


def _f32_to_key(x):
    k = lax.bitcast_convert_type(x, jnp.int32)
    u = lax.bitcast_convert_type(k, jnp.uint32)
    return jnp.where(k < 0, ~u, u | jnp.uint32(0x80000000))


def _threshold_body(pre_ref, t_ref):
    pre = pre_ref[...]

    cm = pre[:, :128]
    for c in range(1, HIDDEN_DIM // 128):
        cm = jnp.maximum(cm, pre[:, c * 128:(c + 1) * 128])
    himax = _f32_to_key(jnp.max(cm, axis=1, keepdims=True))

    def step_cm(_, carry):
        lo, hi = carry
        mid = lo + ((hi - lo + jnp.uint32(1)) >> jnp.uint32(1))
        t_f = _u32_to_f32(mid)
        cnt = jnp.sum((cm >= t_f).astype(jnp.int32), axis=1, keepdims=True)
        take = cnt >= TOPK
        lo = jnp.where(take, mid, lo)
        hi = jnp.where(take, hi, mid - jnp.uint32(1))
        return lo, hi

    lo_cm, _ = lax.fori_loop(
        0, 32, step_cm, (jnp.zeros((TB, 1), jnp.uint32), himax))

    def cond(carry):
        lo, hi = carry
        return jnp.any(lo < hi)

    def step(carry):
        lo, hi = carry
        mid = lo + ((hi - lo + jnp.uint32(1)) >> jnp.uint32(1))
        t_f = _u32_to_f32(mid)
        cnt = jnp.sum((pre >= t_f).astype(jnp.int32), axis=1, keepdims=True)
        take = cnt >= TOPK
        lo = jnp.where(take, mid, lo)
        hi = jnp.where(take, hi, mid - jnp.uint32(1))
        return lo, hi

    lo, _ = lax.while_loop(cond, step, (lo_cm, himax))
    t_ref[...] = jnp.broadcast_to(_u32_to_f32(lo), (TB, 128))


def _threshold(pre):
    return pl.pallas_call(
        _threshold_body,
        grid=(BATCH // TB,),
        in_specs=[pl.BlockSpec((TB, HIDDEN_DIM), lambda b: (b, 0))],
        out_specs=pl.BlockSpec((TB, 128), lambda b: (b, 0)),
        out_shape=jax.ShapeDtypeStruct((BATCH, 128), jnp.float32),
    )(pre)


def _decode_body(pre_ref, t_ref, w_ref, b_ref, out_ref):
    k = pl.program_id(1)
    t = t_ref[...][:, :1]
    p = pre_ref[...]
    z = jnp.where(p >= t, jnp.maximum(p, 0.0), 0.0).astype(jnp.bfloat16)
    acc = jax.lax.dot_general(
        z, w_ref[...].astype(jnp.bfloat16),
        dimension_numbers=(((1,), (1,)), ((), ())),
        preferred_element_type=jnp.float32,
    )

    @pl.when(k == 0)
    def _():
        out_ref[...] = acc + b_ref[...][None, :]

    @pl.when(k > 0)
    def _():
        out_ref[...] += acc


def _masked_decode(pre, trep, W_dec, bias):
    nb = pre.shape[0]
    return pl.pallas_call(
        _decode_body,
        grid=(nb // DB, HIDDEN_DIM // KB),
        in_specs=[
            pl.BlockSpec((DB, KB), lambda b, k: (b, k)),
            pl.BlockSpec((DB, 128), lambda b, k: (b, 0)),
            pl.BlockSpec((INPUT_DIM, KB), lambda b, k: (0, k)),
            pl.BlockSpec((INPUT_DIM,), lambda b, k: (0,)),
        ],
        out_specs=pl.BlockSpec((DB, INPUT_DIM), lambda b, k: (b, 0)),
        out_shape=jax.ShapeDtypeStruct((nb, INPUT_DIM), jnp.float32),
    )(pre, trep, W_dec, bias)


NC = 2
NS = 16
NW = NC * NS
SC_BATCH = 256
ROWS_PER_W = SC_BATCH // NW
L = 16
NB = 8
GCHUNK = 16
NCHUNK = TOPK // GCHUNK
VTILE = 32
NTILE = INPUT_DIM // (VTILE * L)


def _take(v, idx):
    return jnp.take_along_axis(v, idx, axis=0, mode="promise_in_bounds")


def _sc_body(pre_hbm, trep_hbm, w_hbm, bias_hbm, out_hbm,
             pre2, t_v, valbuf, idxbuf, valrep, rows2,
             acc_v, bias_v, sem_pre, sem_g, sem_out):
    wid = lax.axis_index("s") * NC + lax.axis_index("c")
    base_row = wid * ROWS_PER_W

    pltpu.sync_copy(bias_hbm, bias_v)
    pltpu.sync_copy(trep_hbm.at[pl.ds(base_row, ROWS_PER_W)], t_v)
    pltpu.async_copy(pre_hbm.at[base_row], pre2.at[0], sem_pre)
    lane = lax.iota(jnp.int32, L)
    ones = jnp.ones((L,), jnp.float32)
    perms = [lax.rem(lane + l, L) for l in range(L)]
    folds = [jnp.bitwise_xor(lane, s) for s in (8, 4, 2, 1)]

    def _fold_max(x):
        for f in folds:
            x = jnp.maximum(x, _take(x, f))
        return x[0]

    def do_row(r, carry):
        row = base_row + r
        slot = lax.rem(r, 2)
        nslot = lax.rem(r + 1, 2)
        pltpu.make_async_copy(pre_hbm.at[row], pre2.at[slot], sem_pre).wait()

        @pl.when(r < ROWS_PER_W - 1)
        def _():
            pltpu.async_copy(pre_hbm.at[row + 1], pre2.at[nslot], sem_pre)

        tv = t_v[r, :]
        ts = tv[0]

        def batch(i, off):
            mx = pre2[slot, pl.ds(i * (NB * L), L)]
            for q in range(1, NB):
                mx = jnp.maximum(mx, pre2[slot, pl.ds(i * (NB * L) + q * L, L)])

            def hit_batch(off):
                for q in range(NB):
                    v = pre2[slot, pl.ds(i * (NB * L) + q * L, L)]
                    mi = jnp.where(v >= tv, jnp.int32(1), jnp.int32(0))

                    def hit_vreg(off, v=v, mi=mi, q=q):
                        iv = lane + (i * (NB * L) + q * L)

                        def put(off, l=0):
                            pass
                        for l in range(L):
                            def put_l(off, l=l, v=v, iv=iv):
                                valbuf[pl.ds(off, L)] = _take(v, perms[l])
                                idxbuf[pl.ds(off, L)] = _take(iv, perms[l])
                                return off + 1
                            off = lax.cond(
                                jnp.logical_and(mi[l] > 0, off < TOPK),
                                put_l, lambda o: o, off)
                        return off

                    off = lax.cond(_fold_max(mi) > 0, hit_vreg,
                                   lambda o: o, off)
                return off

            return lax.cond(_fold_max(mx) >= ts, hit_batch,
                            lambda o: o, off)

        lax.fori_loop(0, HIDDEN_DIM // (NB * L), batch, jnp.int32(0))

        for c in range(TOPK // L):
            vb = jnp.maximum(valbuf[pl.ds(c * L, L)], 0.0)
            for j in range(L):
                valrep[c * L + j, :] = vb[j] * ones

        pltpu.async_copy(w_hbm.at[idxbuf.at[pl.ds(0, GCHUNK)]],
                         rows2.at[0], sem_g)
        for c in range(NCHUNK):
            gslot = c % 2
            pltpu.make_async_copy(w_hbm.at[idxbuf.at[pl.ds(c * GCHUNK, GCHUNK)]],
                                  rows2.at[gslot], sem_g).wait()
            if c < NCHUNK - 1:
                pltpu.async_copy(
                    w_hbm.at[idxbuf.at[pl.ds((c + 1) * GCHUNK, GCHUNK)]],
                    rows2.at[(c + 1) % 2], sem_g)
            for t in range(NTILE):
                tbase = t * VTILE * L
                if c == 0:
                    acc = tuple(bias_v[pl.ds(tbase + i2 * L, L)]
                                for i2 in range(VTILE))
                else:
                    acc = tuple(acc_v[pl.ds(tbase + i2 * L, L)]
                                for i2 in range(VTILE))

                def mac(j, a, c=c, gslot=gslot, tbase=tbase):
                    sv = valrep[c * GCHUNK + j, :]
                    return tuple(
                        a[i2] + sv * rows2[gslot, j, pl.ds(tbase + i2 * L, L)]
                        for i2 in range(VTILE))

                acc = lax.fori_loop(0, GCHUNK, mac, acc, unroll=4)
                for i2 in range(VTILE):
                    acc_v[pl.ds(tbase + i2 * L, L)] = acc[i2]

        pltpu.async_copy(acc_v, out_hbm.at[row], sem_out).wait()
        return carry

    lax.fori_loop(0, ROWS_PER_W, do_row, 0)


def _sc_topk_decode(pre, trep, W_enc, bias):
    from jax.experimental.pallas import tpu_sc as plsc
    mesh = plsc.VectorSubcoreMesh(core_axis_name="c", subcore_axis_name="s")
    k = pl.kernel(
        _sc_body,
        mesh=mesh,
        out_type=jax.ShapeDtypeStruct((SC_BATCH, INPUT_DIM), jnp.float32),
        scratch_types=[
            pltpu.VMEM((2, HIDDEN_DIM), jnp.float32),
            pltpu.VMEM((ROWS_PER_W, L), jnp.float32),
            pltpu.VMEM((TOPK + L,), jnp.float32),
            pltpu.VMEM((TOPK + L,), jnp.int32),
            pltpu.VMEM((TOPK, L), jnp.float32),
            pltpu.VMEM((2, GCHUNK, INPUT_DIM), jnp.float32),
            pltpu.VMEM((INPUT_DIM,), jnp.float32),
            pltpu.VMEM((INPUT_DIM,), jnp.float32),
            pltpu.SemaphoreType.DMA,
            pltpu.SemaphoreType.DMA,
            pltpu.SemaphoreType.DMA,
        ],
    )
    return k(pre, trep, W_enc, bias)


TC_BATCH = BATCH - SC_BATCH
DB = 896


def kernel(x, W_enc, W_dec, bias):
    pre = _encode(x, W_enc, bias)
    trep = _threshold(pre)
    out_sc = _sc_topk_decode(pre[TC_BATCH:], trep[TC_BATCH:, :L], W_enc, bias)
    out_tc = _masked_decode(pre[:TC_BATCH], trep[:TC_BATCH], W_dec, bias)
    return jnp.concatenate([out_tc, out_sc], axis=0)


# speedup vs baseline: 3.8220x; 1.4620x over previous
import jax
import jax.numpy as jnp
from jax import lax
from jax.experimental import pallas as pl
from jax.experimental.pallas import tpu as pltpu

INPUT_DIM = 2048
HIDDEN_DIM = 16384
TOPK = 64
BATCH = 2048

BH = 512
TB = 256
KB = 1024
DB = 1024


def _encode_body(x_ref, w_ref, b_ref, out_ref):
    xm = (x_ref[...] - b_ref[...][None, :]).astype(jnp.bfloat16)
    out_ref[...] = jax.lax.dot_general(
        xm, w_ref[...].astype(jnp.bfloat16),
        dimension_numbers=(((1,), (1,)), ((), ())),
        preferred_element_type=jnp.float32,
    )


def _encode(x, W_enc, bias):
    return pl.pallas_call(
        _encode_body,
        grid=(HIDDEN_DIM // BH,),
        in_specs=[
            pl.BlockSpec((BATCH, INPUT_DIM), lambda h: (0, 0)),
            pl.BlockSpec((BH, INPUT_DIM), lambda h: (h, 0)),
            pl.BlockSpec((INPUT_DIM,), lambda h: (0,)),
        ],
        out_specs=pl.BlockSpec((BATCH, BH), lambda h: (0, h)),
        out_shape=jax.ShapeDtypeStruct((BATCH, HIDDEN_DIM), jnp.float32),
    )(x, W_enc, bias)


def _u32_to_f32(t):
    neg = (t & jnp.uint32(0x80000000)) == 0
    bits = jnp.where(neg, ~t, t & jnp.uint32(0x7FFFFFFF))
    return lax.bitcast_convert_type(bits, jnp.float32)


def _f32_to_key(x):
    k = lax.bitcast_convert_type(x, jnp.int32)
    u = lax.bitcast_convert_type(k, jnp.uint32)
    return jnp.where(k < 0, ~u, u | jnp.uint32(0x80000000))


def _threshold_body(pre_ref, t_ref):
    pre = pre_ref[...]

    cm = pre[:, :128]
    for c in range(1, HIDDEN_DIM // 128):
        cm = jnp.maximum(cm, pre[:, c * 128:(c + 1) * 128])
    himax = _f32_to_key(jnp.max(cm, axis=1, keepdims=True))

    def step_cm(_, carry):
        lo, hi = carry
        mid = lo + ((hi - lo + jnp.uint32(1)) >> jnp.uint32(1))
        t_f = _u32_to_f32(mid)
        cnt = jnp.sum((cm >= t_f).astype(jnp.int32), axis=1, keepdims=True)
        take = cnt >= TOPK
        lo = jnp.where(take, mid, lo)
        hi = jnp.where(take, hi, mid - jnp.uint32(1))
        return lo, hi

    lo_cm, _ = lax.fori_loop(
        0, 32, step_cm, (jnp.zeros((TB, 1), jnp.uint32), himax))

    def cond(carry):
        lo, hi = carry
        return jnp.any(lo < hi)

    def step(carry):
        lo, hi = carry
        mid = lo + ((hi - lo + jnp.uint32(1)) >> jnp.uint32(1))
        t_f = _u32_to_f32(mid)
        cnt = jnp.sum((pre >= t_f).astype(jnp.int32), axis=1, keepdims=True)
        take = cnt >= TOPK
        found = cnt == TOPK
        lo = jnp.where(found, mid, jnp.where(take, mid, lo))
        hi = jnp.where(found, mid, jnp.where(take, hi, mid - jnp.uint32(1)))
        return lo, hi

    lo, _ = lax.while_loop(cond, step, (lo_cm, himax))
    t_ref[...] = jnp.broadcast_to(_u32_to_f32(lo), (TB, 128))


def _threshold(pre):
    return pl.pallas_call(
        _threshold_body,
        grid=(BATCH // TB,),
        in_specs=[pl.BlockSpec((TB, HIDDEN_DIM), lambda b: (b, 0))],
        out_specs=pl.BlockSpec((TB, 128), lambda b: (b, 0)),
        out_shape=jax.ShapeDtypeStruct((BATCH, 128), jnp.float32),
    )(pre)


def _decode_body(pre_ref, t_ref, w_ref, b_ref, out_ref):
    k = pl.program_id(1)
    t = t_ref[...][:, :1]
    p = pre_ref[...]
    z = jnp.where(p >= t, jnp.maximum(p, 0.0), 0.0).astype(jnp.bfloat16)
    acc = jax.lax.dot_general(
        z, w_ref[...].astype(jnp.bfloat16),
        dimension_numbers=(((1,), (1,)), ((), ())),
        preferred_element_type=jnp.float32,
    )

    @pl.when(k == 0)
    def _():
        out_ref[...] = acc + b_ref[...][None, :]

    @pl.when(k > 0)
    def _():
        out_ref[...] += acc


def _masked_decode(pre, trep, W_dec, bias):
    return pl.pallas_call(
        _decode_body,
        grid=(BATCH // DB, HIDDEN_DIM // KB),
        in_specs=[
            pl.BlockSpec((DB, KB), lambda b, k: (b, k)),
            pl.BlockSpec((DB, 128), lambda b, k: (b, 0)),
            pl.BlockSpec((INPUT_DIM, KB), lambda b, k: (0, k)),
            pl.BlockSpec((INPUT_DIM,), lambda b, k: (0,)),
        ],
        out_specs=pl.BlockSpec((DB, INPUT_DIM), lambda b, k: (b, 0)),
        out_shape=jax.ShapeDtypeStruct((BATCH, INPUT_DIM), jnp.float32),
    )(pre, trep, W_dec, bias)


def kernel(x, W_enc, W_dec, bias):
    pre = _encode(x, W_enc, bias)
    trep = _threshold(pre)
    return _masked_decode(pre, trep, W_dec, bias)
